# baseline (speedup 1.0000x reference)
import functools

import jax
import jax.numpy as jnp
from jax import lax
from jax.experimental import pallas as pl
from jax.experimental.pallas import tpu as pltpu
from jax.experimental.pallas import tpu_sc as plsc

N = 1_000_000
L = 16
H = 2

BT = 131072
XB = 7
X = XB * BT
CH = 2048
C0 = X // CH
NCH = N // CH
TAIL = 512
SCN = NCH * CH - X + TAIL
REST = N - NCH * CH - TAIL
NW = 32
TMAX = (NCH - C0 + NW - 1) // NW


def _make_sc_run():
    mesh = plsc.VectorSubcoreMesh(core_axis_name="c", subcore_axis_name="s")

    @functools.partial(
        pl.kernel,
        mesh=mesh,
        compiler_params=pltpu.CompilerParams(needs_layout_passes=False),
        out_type=jax.ShapeDtypeStruct((H, SCN), jnp.float32),
        scratch_types=[
            pltpu.VMEM((H * L * L,), jnp.float32),
            pltpu.VMEM((L, CH), jnp.float32),
            pltpu.VMEM((H, CH), jnp.float32),
        ],
    )
    def run(xt, w_hbm, ot, w_v, xb, ob):
        cid = lax.axis_index("c")
        sid = lax.axis_index("s")
        wid = sid * 2 + cid

        pltpu.sync_copy(w_hbm, w_v)

        w = [[w_v[pl.ds((h * L + j) * L, L)] for j in range(L)] for h in range(H)]

        def do_chunk(nvec):
            def vec_body(c, carry):
                base = c * L
                a0 = [None] * 4
                a1 = [None] * 4
                for j in range(L):
                    v = xb[j, pl.ds(base, L)]
                    k = j % 4
                    if j < 4:
                        a0[k] = v * w[0][j]
                        a1[k] = v * w[1][j]
                    else:
                        a0[k] = a0[k] + v * w[0][j]
                        a1[k] = a1[k] + v * w[1][j]
                ob[0, pl.ds(base, L)] = (a0[0] + a0[1]) + (a0[2] + a0[3])
                ob[1, pl.ds(base, L)] = (a1[0] + a1[1]) + (a1[2] + a1[3])
                return carry

            lax.fori_loop(0, nvec, vec_body, 0)

        def blk_body(t, carry):
            blk = C0 + wid + t * NW

            @pl.when(blk < NCH)
            def _():
                pltpu.sync_copy(xt.at[:, pl.ds(blk * CH, CH)], xb)
                do_chunk(CH // L)
                pltpu.sync_copy(ob, ot.at[:, pl.ds(blk * CH - X, CH)])

            return carry

        lax.fori_loop(0, TMAX, blk_body, 0)

        @pl.when(wid == NW - 1)
        def _():
            a0 = NCH * CH
            pltpu.sync_copy(xt.at[:, pl.ds(a0, TAIL)], xb.at[:, pl.ds(0, TAIL)])
            do_chunk(TAIL // L)
            pltpu.sync_copy(ob.at[:, pl.ds(0, TAIL)], ot.at[:, pl.ds(a0 - X, TAIL)])

    return run


_sc_run = _make_sc_run()


def _tc_body(w_ref, x_ref, o_ref):
    o_ref[...] = jax.lax.dot_general(
        w_ref[...], x_ref[...], (((1,), (0,)), ((), ())),
        preferred_element_type=jnp.float32,
    )


_tc_run = pl.pallas_call(
    _tc_body,
    grid=(XB,),
    in_specs=[
        pl.BlockSpec((H, L), lambda i: (0, 0)),
        pl.BlockSpec((L, BT), lambda i: (0, i)),
    ],
    out_specs=pl.BlockSpec((H, BT), lambda i: (0, i)),
    out_shape=jax.ShapeDtypeStruct((H, N), jnp.float32),
)


def kernel(x, atomic_energy):
    xt = x.T
    wb = jnp.broadcast_to(atomic_energy[:, :, None], (H, L, L))
    sc_out = _sc_run(xt, wb.reshape(H * L * L))
    out_t = _tc_run(atomic_energy, xt)
    out_t = lax.dynamic_update_slice(out_t, sc_out, (0, X))
    tail_t = atomic_energy @ x[N - REST :, :].T
    out_t = lax.dynamic_update_slice(out_t, tail_t, (0, N - REST))
    return out_t.T


# speedup vs baseline: 1.0085x; 1.0085x over previous
import functools

import jax
import jax.numpy as jnp
from jax import lax
from jax.experimental import pallas as pl
from jax.experimental.pallas import tpu as pltpu
from jax.experimental.pallas import tpu_sc as plsc

N = 1_000_000
L = 16
H = 2

BT = 131072
XB = 7
X = XB * BT
CH = 2048
C0 = X // CH
NCH = N // CH
TAIL = 512
SCN = NCH * CH - X + TAIL
REST = N - NCH * CH - TAIL
NW = 32
TMAX = (NCH - C0 + NW - 1) // NW


def _make_sc_run():
    mesh = plsc.VectorSubcoreMesh(core_axis_name="c", subcore_axis_name="s")

    @functools.partial(
        pl.kernel,
        mesh=mesh,
        compiler_params=pltpu.CompilerParams(needs_layout_passes=False),
        out_type=jax.ShapeDtypeStruct((H, SCN), jnp.float32),
        scratch_types=[
            pltpu.VMEM((H * L * L,), jnp.float32),
            pltpu.VMEM((L, CH), jnp.float32),
            pltpu.VMEM((H, CH), jnp.float32),
        ],
    )
    def run(xt, w_hbm, ot, w_v, xb, ob):
        cid = lax.axis_index("c")
        sid = lax.axis_index("s")
        wid = sid * 2 + cid

        pltpu.sync_copy(w_hbm, w_v)

        w = [[w_v[pl.ds((h * L + j) * L, L)] for j in range(L)] for h in range(H)]

        def do_chunk(nvec):
            def vec_body(c, carry):
                base = c * L
                a0 = [None] * 4
                a1 = [None] * 4
                for j in range(L):
                    v = xb[j, pl.ds(base, L)]
                    k = j % 4
                    if j < 4:
                        a0[k] = v * w[0][j]
                        a1[k] = v * w[1][j]
                    else:
                        a0[k] = a0[k] + v * w[0][j]
                        a1[k] = a1[k] + v * w[1][j]
                ob[0, pl.ds(base, L)] = (a0[0] + a0[1]) + (a0[2] + a0[3])
                ob[1, pl.ds(base, L)] = (a1[0] + a1[1]) + (a1[2] + a1[3])
                return carry

            lax.fori_loop(0, nvec, vec_body, 0)

        def blk_body(t, carry):
            blk = C0 + wid + t * NW

            @pl.when(blk < NCH)
            def _():
                pltpu.sync_copy(xt.at[:, pl.ds(blk * CH, CH)], xb)
                do_chunk(CH // L)
                pltpu.sync_copy(ob, ot.at[:, pl.ds(blk * CH - X, CH)])

            return carry

        lax.fori_loop(0, TMAX, blk_body, 0)

        @pl.when(wid == NW - 1)
        def _():
            a0 = NCH * CH
            pltpu.sync_copy(xt.at[:, pl.ds(a0, TAIL)], xb.at[:, pl.ds(0, TAIL)])
            do_chunk(TAIL // L)
            pltpu.sync_copy(ob.at[:, pl.ds(0, TAIL)], ot.at[:, pl.ds(a0 - X, TAIL)])

    return run


_sc_run = _make_sc_run()


def _tc_body(w_ref, x_ref, o_ref):
    o_ref[...] = jax.lax.dot_general(
        w_ref[...], x_ref[...], (((1,), (0,)), ((), ())),
        preferred_element_type=jnp.float32,
    )


_tc_run = pl.pallas_call(
    _tc_body,
    grid=(XB,),
    in_specs=[
        pl.BlockSpec((H, L), lambda i: (0, 0)),
        pl.BlockSpec((L, BT), lambda i: (0, i)),
    ],
    out_specs=pl.BlockSpec((H, BT), lambda i: (0, i)),
    out_shape=jax.ShapeDtypeStruct((H, N), jnp.float32),
)


def kernel(x, atomic_energy):
    xt = x.T
    wb = jnp.repeat(atomic_energy.reshape(H * L), L)
    sc_out = _sc_run(xt, wb)
    out_t = _tc_run(atomic_energy, xt)
    tail_t = atomic_energy @ x[N - REST :, :].T
    upd = jnp.concatenate([sc_out, tail_t], axis=1)
    out_t = lax.dynamic_update_slice(out_t, upd, (0, X))
    return out_t.T


# speedup vs baseline: 1.0161x; 1.0075x over previous
import functools

import jax
import jax.numpy as jnp
from jax import lax
from jax.experimental import pallas as pl
from jax.experimental.pallas import tpu as pltpu
from jax.experimental.pallas import tpu_sc as plsc

N = 1_000_000
L = 16
H = 2

BT = 131072
XB = 7
X = XB * BT
CH = 2048
C0 = X // CH
NCH = N // CH
TAIL = 512
SCN = NCH * CH - X + TAIL
REST = N - NCH * CH - TAIL
NW = 32
TMAX = (NCH - C0 + NW - 1) // NW


def _make_sc_run():
    mesh = plsc.VectorSubcoreMesh(core_axis_name="c", subcore_axis_name="s")

    @functools.partial(
        pl.kernel,
        mesh=mesh,
        compiler_params=pltpu.CompilerParams(needs_layout_passes=False),
        out_type=jax.ShapeDtypeStruct((H, SCN), jnp.float32),
        scratch_types=[
            pltpu.VMEM((H * L,), jnp.float32),
            pltpu.VMEM((L, CH), jnp.float32),
            pltpu.VMEM((H, CH), jnp.float32),
        ],
    )
    def run(xt, w_hbm, ot, w_v, xb, ob):
        cid = lax.axis_index("c")
        sid = lax.axis_index("s")
        wid = sid * 2 + cid

        pltpu.sync_copy(w_hbm, w_v)

        w = [
            [
                plsc.load_gather(w_v, [jnp.full((L,), h * L + j, jnp.int32)])
                for j in range(L)
            ]
            for h in range(H)
        ]

        def do_chunk(nvec):
            def vec_body(c, carry):
                base = c * L
                a0 = [None] * 4
                a1 = [None] * 4
                for j in range(L):
                    v = xb[j, pl.ds(base, L)]
                    k = j % 4
                    if j < 4:
                        a0[k] = v * w[0][j]
                        a1[k] = v * w[1][j]
                    else:
                        a0[k] = a0[k] + v * w[0][j]
                        a1[k] = a1[k] + v * w[1][j]
                ob[0, pl.ds(base, L)] = (a0[0] + a0[1]) + (a0[2] + a0[3])
                ob[1, pl.ds(base, L)] = (a1[0] + a1[1]) + (a1[2] + a1[3])
                return carry

            lax.fori_loop(0, nvec, vec_body, 0)

        def blk_body(t, carry):
            blk = C0 + wid + t * NW

            @pl.when(blk < NCH)
            def _():
                pltpu.sync_copy(xt.at[:, pl.ds(blk * CH, CH)], xb)
                do_chunk(CH // L)
                pltpu.sync_copy(ob, ot.at[:, pl.ds(blk * CH - X, CH)])

            return carry

        lax.fori_loop(0, TMAX, blk_body, 0)

        @pl.when(wid == NW - 1)
        def _():
            a0 = NCH * CH
            pltpu.sync_copy(xt.at[:, pl.ds(a0, TAIL)], xb.at[:, pl.ds(0, TAIL)])
            do_chunk(TAIL // L)
            pltpu.sync_copy(ob.at[:, pl.ds(0, TAIL)], ot.at[:, pl.ds(a0 - X, TAIL)])

    return run


_sc_run = _make_sc_run()


def _tc_body(w_ref, x_ref, o_ref):
    o_ref[...] = jax.lax.dot_general(
        w_ref[...], x_ref[...], (((1,), (0,)), ((), ())),
        preferred_element_type=jnp.float32,
    )


_tc_run = pl.pallas_call(
    _tc_body,
    grid=(XB,),
    in_specs=[
        pl.BlockSpec((H, L), lambda i: (0, 0)),
        pl.BlockSpec((L, BT), lambda i: (0, i)),
    ],
    out_specs=pl.BlockSpec((H, BT), lambda i: (0, i)),
    out_shape=jax.ShapeDtypeStruct((H, N), jnp.float32),
)


def kernel(x, atomic_energy):
    xt = x.T
    sc_out = _sc_run(xt, atomic_energy.reshape(H * L))
    out_t = _tc_run(atomic_energy, xt)
    tail_t = atomic_energy @ x[N - REST :, :].T
    upd = jnp.concatenate([sc_out, tail_t], axis=1)
    out_t = lax.dynamic_update_slice(out_t, upd, (0, X))
    return out_t.T
